# baseline (speedup 1.0000x reference)
import functools

import jax
import jax.numpy as jnp
from jax import lax
from jax.experimental import pallas as pl
from jax.experimental.pallas import tpu as pltpu
from jax.experimental.pallas import tpu_sc as plsc

_D = 64
_B = 4096 * 200
_NC, _NS = 2, 16
_NW = _NC * _NS
_BPW = _B // _NW
_C = 128
_NCHUNK = _BPW // _C


def _sc_gather(idx_flat, weight):
    mesh = plsc.VectorSubcoreMesh(core_axis_name="c", subcore_axis_name="s")

    @functools.partial(
        pl.kernel,
        out_type=jax.ShapeDtypeStruct((_B, _D), jnp.float32),
        mesh=mesh,
        scratch_types=[
            pltpu.VMEM((_BPW,), jnp.int32),
            pltpu.VMEM((_C, _D), jnp.float32),
            pltpu.SemaphoreType.DMA,
        ],
        compiler_params=pltpu.CompilerParams(use_tc_tiling_on_sc=False),
    )
    def k(weight_hbm, idx_hbm, out_hbm, idx_v, rows_v, sem):
        wid = lax.axis_index("s") * _NC + lax.axis_index("c")
        base = wid * _BPW
        pltpu.sync_copy(idx_hbm.at[pl.ds(base, _BPW)], idx_v)

        @pl.loop(0, _NCHUNK)
        def _body(g):
            idx_slice = idx_v.at[pl.ds(g * _C, _C)]
            pltpu.async_copy(weight_hbm.at[idx_slice], rows_v, sem).wait()
            pltpu.sync_copy(rows_v, out_hbm.at[pl.ds(base + g * _C, _C)])

    return k(weight, idx_flat)


def kernel(input_ids, weight):
    idx = input_ids.reshape(-1).astype(jnp.int32)
    out = _sc_gather(idx, weight)
    return out.reshape(*input_ids.shape, _D)


# speedup vs baseline: 1.1143x; 1.1143x over previous
import functools

import jax
import jax.numpy as jnp
from jax import lax
from jax.experimental import pallas as pl
from jax.experimental.pallas import tpu as pltpu
from jax.experimental.pallas import tpu_sc as plsc

_D = 64
_B = 4096 * 200
_NC, _NS = 2, 16
_NW = _NC * _NS
_BPW = _B // _NW
_C = 128
_NCHUNK = _BPW // _C
_NBUF = 8
_INFLIGHT = 4


def _sc_gather(idx_flat, weight):
    mesh = plsc.VectorSubcoreMesh(core_axis_name="c", subcore_axis_name="s")

    @functools.partial(
        pl.kernel,
        out_type=jax.ShapeDtypeStruct((_B, _D), jnp.float32),
        mesh=mesh,
        scratch_types=[
            pltpu.VMEM((_BPW,), jnp.int32),
            pltpu.VMEM((_NBUF, _C, _D), jnp.float32),
            [pltpu.SemaphoreType.DMA] * _NBUF,
            [pltpu.SemaphoreType.DMA] * _NBUF,
        ],
        compiler_params=pltpu.CompilerParams(use_tc_tiling_on_sc=False),
    )
    def k(weight_hbm, idx_hbm, out_hbm, idx_v, rows_v, gsem, wsem):
        wid = lax.axis_index("s") * _NC + lax.axis_index("c")
        base = wid * _BPW
        pltpu.sync_copy(idx_hbm.at[pl.ds(base, _BPW)], idx_v)

        def g_src(g):
            return weight_hbm.at[idx_v.at[pl.ds(g * _C, _C)]]

        def w_dst(g):
            return out_hbm.at[pl.ds(base + g * _C, _C)]

        def gstart(g, b):
            pltpu.async_copy(g_src(g), rows_v.at[b], gsem[b])

        def gwait(g, b):
            pltpu.make_async_copy(g_src(g), rows_v.at[b], gsem[b]).wait()

        def wstart(g, b):
            pltpu.async_copy(rows_v.at[b], w_dst(g), wsem[b])

        def wwait(g, b):
            pltpu.make_async_copy(rows_v.at[b], w_dst(g), wsem[b]).wait()

        for i in range(_INFLIGHT):
            gstart(i, i)

        @pl.loop(0, _NCHUNK, step=_NBUF)
        def _outer(g0):
            for b in range(_NBUF):
                g = g0 + b
                gwait(g, b)
                wstart(g, b)
                nxt = g + _INFLIGHT
                b2 = (b + _INFLIGHT) % _NBUF

                @pl.when(nxt < _NCHUNK)
                def _():
                    prev = nxt - _NBUF

                    @pl.when(prev >= 0)
                    def _():
                        wwait(prev, b2)

                    gstart(nxt, b2)

        for b in range(_NBUF):
            wwait(_NCHUNK - _NBUF + b, b)

    return k(weight, idx_flat)


def kernel(input_ids, weight):
    idx = input_ids.reshape(-1).astype(jnp.int32)
    out = _sc_gather(idx, weight)
    return out.reshape(*input_ids.shape, _D)
